# Initial kernel scaffold; baseline (speedup 1.0000x reference)
import jax
import jax.numpy as jnp
from jax.experimental import pallas as pl


def kernel(in_field, types, scales, shifts):
    raise NotImplementedError("write your pallas kernel here")


# speedup vs baseline: 3.7045x; 3.7045x over previous
import functools

import jax
import jax.numpy as jnp
from jax import lax
from jax.experimental import pallas as pl
from jax.experimental.pallas import tpu as pltpu
from jax.experimental.pallas import tpu_sc as plsc

_LANES = 16
_NUM_WORKERS = 32


@functools.lru_cache(maxsize=None)
def _build(n: int, num_types: int):
    chunk = ((n + _NUM_WORKERS - 1) // _NUM_WORKERS + _LANES - 1) // _LANES * _LANES
    assert (n - chunk) % 8 == 0 and n >= chunk
    n_vecs = chunk // _LANES

    mesh = plsc.VectorSubcoreMesh(core_axis_name="c", subcore_axis_name="s")

    @functools.partial(
        pl.kernel,
        mesh=mesh,
        compiler_params=pltpu.CompilerParams(needs_layout_passes=False),
        out_type=jax.ShapeDtypeStruct((n,), jnp.float32),
        scratch_types=[
            pltpu.VMEM((chunk,), jnp.float32),
            pltpu.VMEM((chunk,), jnp.int32),
            pltpu.VMEM((chunk,), jnp.float32),
            pltpu.VMEM((num_types,), jnp.float32),
            pltpu.VMEM((num_types,), jnp.float32),
        ],
    )
    def scale_shift(x_hbm, t_hbm, s_hbm, b_hbm, out_hbm, xv, tv, ov, sv, bv):
        wid = lax.axis_index("s") * 2 + lax.axis_index("c")
        base = jnp.minimum(wid * chunk, n - chunk)
        pltpu.sync_copy(s_hbm, sv)
        pltpu.sync_copy(b_hbm, bv)
        pltpu.sync_copy(x_hbm.at[pl.ds(base, chunk)], xv)
        pltpu.sync_copy(t_hbm.at[pl.ds(base, chunk)], tv)

        def step(i, carry):
            sl = pl.ds(i * _LANES, _LANES)
            tt = tv[sl]
            xx = xv[sl]
            ss = plsc.load_gather(sv, [tt])
            bb = plsc.load_gather(bv, [tt])
            ov[sl] = bb + ss * xx
            return carry

        lax.fori_loop(0, n_vecs, step, 0)
        pltpu.sync_copy(ov, out_hbm.at[pl.ds(base, chunk)])

    return scale_shift


def kernel(in_field, types, scales, shifts):
    n = in_field.shape[0]
    num_types = scales.shape[0]
    fn = _build(n, num_types)
    out = fn(
        in_field.reshape(n),
        types.reshape(n).astype(jnp.int32),
        scales.reshape(num_types),
        shifts.reshape(num_types),
    )
    return out.reshape(n, 1)


# speedup vs baseline: 3.7832x; 1.0212x over previous
import functools

import jax
import jax.numpy as jnp
from jax import lax
from jax.experimental import pallas as pl
from jax.experimental.pallas import tpu as pltpu
from jax.experimental.pallas import tpu_sc as plsc

_LANES = 16
_NUM_WORKERS = 32


@functools.lru_cache(maxsize=None)
def _build(n: int, num_types: int):
    unroll = 8
    grain = _LANES * unroll
    chunk = ((n + _NUM_WORKERS - 1) // _NUM_WORKERS + grain - 1) // grain * grain
    assert (n - chunk) % 8 == 0 and n >= chunk

    mesh = plsc.VectorSubcoreMesh(core_axis_name="c", subcore_axis_name="s")

    @functools.partial(
        pl.kernel,
        mesh=mesh,
        compiler_params=pltpu.CompilerParams(needs_layout_passes=False),
        out_type=jax.ShapeDtypeStruct((n,), jnp.float32),
        scratch_types=[
            pltpu.VMEM((chunk,), jnp.float32),
            pltpu.VMEM((chunk,), jnp.int32),
            pltpu.VMEM((chunk,), jnp.float32),
            pltpu.VMEM((num_types,), jnp.float32),
            pltpu.VMEM((num_types,), jnp.float32),
            pltpu.SemaphoreType.DMA,
        ],
    )
    def scale_shift(x_hbm, t_hbm, s_hbm, b_hbm, out_hbm, xv, tv, ov, sv, bv, sem):
        wid = lax.axis_index("s") * 2 + lax.axis_index("c")
        base = jnp.minimum(wid * chunk, n - chunk)
        c1 = pltpu.async_copy(x_hbm.at[pl.ds(base, chunk)], xv, sem)
        c2 = pltpu.async_copy(t_hbm.at[pl.ds(base, chunk)], tv, sem)
        c3 = pltpu.async_copy(s_hbm, sv, sem)
        c4 = pltpu.async_copy(b_hbm, bv, sem)
        c1.wait()
        c2.wait()
        c3.wait()
        c4.wait()

        @plsc.parallel_loop(0, chunk, step=_LANES, unroll=unroll)
        def _(i):
            sl = pl.ds(i, _LANES)
            tt = tv[sl]
            ss = plsc.load_gather(sv, [tt])
            bb = plsc.load_gather(bv, [tt])
            ov[sl] = bb + ss * xv[sl]

        pltpu.sync_copy(ov, out_hbm.at[pl.ds(base, chunk)])

    return scale_shift


def kernel(in_field, types, scales, shifts):
    n = in_field.shape[0]
    num_types = scales.shape[0]
    fn = _build(n, num_types)
    out = fn(
        in_field.reshape(n),
        types.reshape(n).astype(jnp.int32),
        scales.reshape(num_types),
        shifts.reshape(num_types),
    )
    return out.reshape(n, 1)


# speedup vs baseline: 4.0581x; 1.0727x over previous
import functools

import jax
import jax.numpy as jnp
from jax import lax
from jax.experimental import pallas as pl
from jax.experimental.pallas import tpu as pltpu
from jax.experimental.pallas import tpu_sc as plsc

_LANES = 16
_NUM_WORKERS = 32
_UNROLL = 8


@functools.lru_cache(maxsize=None)
def _build(n: int, num_types: int):
    grain = 2 * _LANES * _UNROLL
    chunk = ((n + _NUM_WORKERS - 1) // _NUM_WORKERS + grain - 1) // grain * grain
    sub = chunk // 2
    assert (n - chunk) % 8 == 0 and n >= chunk

    mesh = plsc.VectorSubcoreMesh(core_axis_name="c", subcore_axis_name="s")

    @functools.partial(
        pl.kernel,
        mesh=mesh,
        compiler_params=pltpu.CompilerParams(needs_layout_passes=False),
        out_type=jax.ShapeDtypeStruct((n,), jnp.float32),
        scratch_types=[
            pltpu.VMEM((sub,), jnp.float32),
            pltpu.VMEM((sub,), jnp.float32),
            pltpu.VMEM((sub,), jnp.int32),
            pltpu.VMEM((sub,), jnp.int32),
            pltpu.VMEM((sub,), jnp.float32),
            pltpu.VMEM((sub,), jnp.float32),
            pltpu.VMEM((num_types,), jnp.float32),
            pltpu.VMEM((num_types,), jnp.float32),
            pltpu.SemaphoreType.DMA,
            pltpu.SemaphoreType.DMA,
            pltpu.SemaphoreType.DMA,
        ],
    )
    def scale_shift(
        x_hbm, t_hbm, s_hbm, b_hbm, out_hbm,
        xv0, xv1, tv0, tv1, ov0, ov1, sv, bv, sem0, sem1, semo,
    ):
        wid = lax.axis_index("s") * 2 + lax.axis_index("c")
        base = jnp.minimum(wid * chunk, n - chunk)
        cs = pltpu.async_copy(s_hbm, sv, sem0)
        cb = pltpu.async_copy(b_hbm, bv, sem0)
        cx0 = pltpu.async_copy(x_hbm.at[pl.ds(base, sub)], xv0, sem0)
        ct0 = pltpu.async_copy(t_hbm.at[pl.ds(base, sub)], tv0, sem0)
        cx1 = pltpu.async_copy(x_hbm.at[pl.ds(base + sub, sub)], xv1, sem1)
        ct1 = pltpu.async_copy(t_hbm.at[pl.ds(base + sub, sub)], tv1, sem1)
        cs.wait()
        cb.wait()
        cx0.wait()
        ct0.wait()

        @plsc.parallel_loop(0, sub, step=_LANES, unroll=_UNROLL)
        def _(i):
            sl = pl.ds(i, _LANES)
            tt = tv0[sl]
            ss = plsc.load_gather(sv, [tt])
            bb = plsc.load_gather(bv, [tt])
            ov0[sl] = bb + ss * xv0[sl]

        co0 = pltpu.async_copy(ov0, out_hbm.at[pl.ds(base, sub)], semo)
        cx1.wait()
        ct1.wait()

        @plsc.parallel_loop(0, sub, step=_LANES, unroll=_UNROLL)
        def _(i):
            sl = pl.ds(i, _LANES)
            tt = tv1[sl]
            ss = plsc.load_gather(sv, [tt])
            bb = plsc.load_gather(bv, [tt])
            ov1[sl] = bb + ss * xv1[sl]

        co1 = pltpu.async_copy(ov1, out_hbm.at[pl.ds(base + sub, sub)], semo)
        co0.wait()
        co1.wait()

    return scale_shift


def kernel(in_field, types, scales, shifts):
    n = in_field.shape[0]
    num_types = scales.shape[0]
    fn = _build(n, num_types)
    types = types.reshape(n)
    if types.dtype != jnp.int32:
        types = types.astype(jnp.int32)
    out = fn(
        in_field.reshape(n),
        types,
        scales.reshape(num_types),
        shifts.reshape(num_types),
    )
    return out.reshape(n, 1)
